# baseline (speedup 1.0000x reference)
import functools

import jax
import jax.numpy as jnp
from jax import lax
from jax.experimental import pallas as pl
from jax.experimental.pallas import tpu as pltpu
from jax.experimental.pallas import tpu_sc as plsc

VOCAB1 = 100001
EMBED_DIM = 32
BATCH = 16384

_info = plsc.get_sparse_core_info()
_NC = _info.num_cores
_NS = _info.num_subcores

_CI = EMBED_DIM // 8
_RJ = (VOCAB1 + 127) // 128
_CPW = (_RJ + _NS - 1) // _NS
_KC = 13
_NCH = (_CPW + _KC - 1) // _KC
_HALF = BATCH // _NC
_SG = _HALF // 16
_DUMP = _HALF + 8


@functools.partial(
    pl.kernel,
    mesh=plsc.VectorSubcoreMesh(core_axis_name="c", subcore_axis_name="s"),
    out_type=jax.ShapeDtypeStruct((EMBED_DIM, BATCH), jnp.float32),
    scratch_types=[
        pltpu.VMEM((_HALF,), jnp.int32),
        pltpu.VMEM((_HALF + 16,), jnp.int32),
        pltpu.VMEM((_HALF + 16,), jnp.int32),
        pltpu.VMEM((32, 128 * _KC), jnp.float32),
        pltpu.VMEM((32, VOCAB1 - 128 * (_RJ - 1)), jnp.float32),
        pltpu.VMEM((16, 32), jnp.float32),
        pltpu.VMEM((16,), jnp.int32),
        pltpu.VMEM((128, 32), jnp.float32),
        pltpu.VMEM((_CI, 8, 128), jnp.float32),
        pltpu.VMEM((_CI, 8, 128), jnp.float32),
        pltpu.VMEM_SHARED((_HALF + 16, 32), jnp.float32),
        pltpu.SemaphoreType.DMA,
        pltpu.SemaphoreType.DMA,
    ],
    compiler_params=pltpu.CompilerParams(
        use_tc_tiling_on_sc=False, disable_bounds_checks=True,
        needs_layout_passes=False,
    ),
)
def _gather_sc_t(table_hbm, idx_hbm, out_hbm, idx_v, rlist, ilist, chunkbuf,
                 tailbuf, rowbuf, i2grp, tbuf, obuf0, obuf1, shared, sem, sem2):
    sc = lax.axis_index("c")
    s = lax.axis_index("s")
    iota = lax.iota(jnp.int32, 16)

    pltpu.sync_copy(idx_hbm.at[pl.ds(sc * _HALF, _HALF)], idx_v)
    pltpu.sync_copy(table_hbm.at[:, pl.ds(128 * (_RJ - 1), VOCAB1 - 128 * (_RJ - 1))], tailbuf)

    def prefill(k, carry):
        rlist[pl.ds(16 * k, 16)] = jnp.full((16,), -1, jnp.int32)
        ilist[pl.ds(16 * k, 16)] = jnp.full((16,), _DUMP, jnp.int32)
        return carry

    lax.fori_loop(0, (_HALF + 16) // 16, prefill, jnp.int32(0))

    lo_col = _CPW * s
    hi_col = jnp.minimum(_CPW * (s + 1), _RJ)
    rlo = 128 * lo_col
    rhi = 128 * hi_col

    def scan(g, cnt):
        v = idx_v[pl.ds(16 * g, 16)]
        m = (v >= rlo) & (v < rhi)
        mi = jnp.where(m, 1, 0)
        dest = cnt + plsc.cumsum(mi) - 1
        plsc.store_scatter(rlist, [dest], v, mask=m)
        plsc.store_scatter(ilist, [dest], iota + 16 * g, mask=m)
        return cnt + jnp.sum(mi)

    cnt = lax.fori_loop(0, _SG, scan, jnp.int32(0))
    gmax = (cnt + 15) // 16

    for ch in range(_NCH):
        c0 = lo_col + _KC * ch
        cstart = jnp.minimum(c0, _RJ - _KC - 1)
        pltpu.async_copy(
            table_hbm.at[:, pl.ds(128 * cstart, 128 * _KC)],
            chunkbuf, sem).wait()
        c1 = jnp.minimum(c0 + _KC, hi_col)

        def extract(g, carry):
            rvec = rlist[pl.ds(16 * g, 16)]
            ivec = ilist[pl.ds(16 * g, 16)]
            rj = lax.shift_right_logical(rvec, 7)
            inch = (rj >= c0) & (rj < c1) & (rj != _RJ - 1)
            iw = jnp.where(inch, ivec, _DUMP)
            jloc = jnp.clip(rj - cstart, 0, _KC - 1)
            tloc = rvec & 127
            for cc in range(EMBED_DIM):
                vals = plsc.load_gather(
                    chunkbuf,
                    [jnp.full((16,), cc, jnp.int32), jloc * 128 + tloc],
                    mask=inch)
                plsc.store_scatter(
                    rowbuf, [iota, jnp.full((16,), cc, jnp.int32)], vals,
                    mask=inch)
            i2grp[...] = iw
            pltpu.async_copy(rowbuf, shared.at[i2grp], sem2).wait()
            return carry

        lax.fori_loop(0, gmax, extract, jnp.int32(0))

    def extract_tail(g, carry):
        rvec = rlist[pl.ds(16 * g, 16)]
        ivec = ilist[pl.ds(16 * g, 16)]
        rj = lax.shift_right_logical(rvec, 7)
        mt = rj == _RJ - 1
        iw = jnp.where(mt, ivec, _DUMP)
        tloc = jnp.clip(rvec - 128 * (_RJ - 1), 0, VOCAB1 - 128 * (_RJ - 1) - 1)
        for cc in range(EMBED_DIM):
            vals = plsc.load_gather(
                tailbuf,
                [jnp.full((16,), cc, jnp.int32), tloc],
                mask=mt)
            plsc.store_scatter(
                rowbuf, [iota, jnp.full((16,), cc, jnp.int32)], vals,
                mask=mt)
        i2grp[...] = iw
        pltpu.async_copy(rowbuf, shared.at[i2grp], sem2).wait()
        return carry

    is_last = jnp.where(hi_col == _RJ, 1, 0)
    lax.fori_loop(0, gmax * is_last, extract_tail, jnp.int32(0))

    plsc.subcore_barrier()

    pend = {}
    for sb in range(4):
        obuf = obuf0 if sb % 2 == 0 else obuf1
        if sb - 2 in pend:
            for h in pend.pop(sb - 2):
                h.wait()
        pltpu.sync_copy(shared.at[pl.ds(512 * s + 128 * sb, 128)], tbuf)
        for ci in range(_CI):
            for sub in range(8):
                cfull = jnp.full((16,), 8 * ci + sub, jnp.int32)
                for tg in range(8):
                    vals = plsc.load_gather(tbuf, [iota + 16 * tg, cfull])
                    obuf[ci, sub, pl.ds(16 * tg, 16)] = vals
        gcol = _HALF * sc + 512 * s + 128 * sb
        pend[sb] = [
            pltpu.async_copy(
                obuf.at[ci],
                out_hbm.at[pl.ds(8 * ci, 8), pl.ds(gcol, 128)], sem)
            for ci in range(_CI)
        ]
    for hs in pend.values():
        for h in hs:
            h.wait()


def kernel(nombre, table):
    idx = nombre.astype(jnp.int32)
    out_t = _gather_sc_t(table.T, idx)
    return out_t.T


# speedup vs baseline: 1.4689x; 1.4689x over previous
import functools

import jax
import jax.numpy as jnp
from jax import lax
from jax.experimental import pallas as pl
from jax.experimental.pallas import tpu as pltpu
from jax.experimental.pallas import tpu_sc as plsc

VOCAB1 = 100001
EMBED_DIM = 32
BATCH = 16384

_info = plsc.get_sparse_core_info()
_NC = _info.num_cores
_NS = _info.num_subcores
_NW = _NC * _NS
_B_PER_W = BATCH // _NW


@functools.partial(
    pl.kernel,
    mesh=plsc.VectorSubcoreMesh(core_axis_name="c", subcore_axis_name="s"),
    out_type=jax.ShapeDtypeStruct((BATCH, EMBED_DIM), jnp.float32),
    scratch_types=[
        pltpu.VMEM((_B_PER_W,), jnp.int32),
        pltpu.VMEM((_B_PER_W, EMBED_DIM), jnp.float32),
        pltpu.SemaphoreType.DMA,
    ],
    compiler_params=pltpu.CompilerParams(use_tc_tiling_on_sc=False),
)
def _gather_sc(table_hbm, idx_hbm, out_hbm, idx_v, rows_v, sem):
    wid = lax.axis_index("s") * _NC + lax.axis_index("c")
    base = wid * _B_PER_W
    pltpu.sync_copy(idx_hbm.at[pl.ds(base, _B_PER_W)], idx_v)
    pltpu.async_copy(table_hbm.at[idx_v], rows_v, sem).wait()
    pltpu.sync_copy(rows_v, out_hbm.at[pl.ds(base, _B_PER_W)])


def kernel(nombre, table):
    idx = nombre.astype(jnp.int32)
    return _gather_sc(table, idx)
